# Initial kernel scaffold; baseline (speedup 1.0000x reference)
import jax
import jax.numpy as jnp
from jax.experimental import pallas as pl


def kernel(x, edge_index, W1, as1, ad1, b1, W2, as2, ad2, b2, W3, as3, ad3, b3, Wsk, bsk):
    raise NotImplementedError("write your pallas kernel here")


# speedup vs baseline: 1.0248x; 1.0248x over previous
import functools

import jax
import jax.numpy as jnp
from jax.experimental import pallas as pl
from jax.experimental.pallas import tpu as pltpu


def _mm_body(x_ref, w_ref, o_ref):
    o_ref[...] = jnp.dot(x_ref[...], w_ref[...],
                         preferred_element_type=jnp.float32)


@functools.partial(jax.jit, static_argnames=("block_m",))
def _matmul(x, w, block_m=1000):
    m, k = x.shape
    k2, n = w.shape
    grid = (m // block_m,)
    return pl.pallas_call(
        _mm_body,
        grid=grid,
        in_specs=[
            pl.BlockSpec((block_m, k), lambda i: (i, 0)),
            pl.BlockSpec((k, n), lambda i: (0, 0)),
        ],
        out_specs=pl.BlockSpec((block_m, n), lambda i: (i, 0)),
        out_shape=jax.ShapeDtypeStruct((m, n), jnp.float32),
    )(x, w)


def _gat_layer(x, src, dst, W, a_s, a_d, b, heads, out_ch):
    N = x.shape[0]
    h = _matmul(x, W).reshape(N, heads, out_ch)
    alpha_src = (h * a_s[None]).sum(-1)
    alpha_dst = (h * a_d[None]).sum(-1)
    alpha = jax.nn.leaky_relu(alpha_src[src] + alpha_dst[dst],
                              negative_slope=0.2)
    amax = jax.ops.segment_max(alpha, dst, num_segments=N)
    ex = jnp.exp(alpha - amax[dst])
    denom = jax.ops.segment_sum(ex, dst, num_segments=N)
    coef = ex / (denom[dst] + 1e-16)
    msg = h[src] * coef[:, :, None]
    out = jax.ops.segment_sum(msg, dst, num_segments=N)
    return out.reshape(N, heads * out_ch) + b


def kernel(x, edge_index, W1, as1, ad1, b1, W2, as2, ad2, b2,
           W3, as3, ad3, b3, Wsk, bsk):
    N = x.shape[0]
    loops = jnp.arange(N, dtype=edge_index.dtype)
    src = jnp.concatenate([edge_index[0], loops])
    dst = jnp.concatenate([edge_index[1], loops])
    identity = jax.nn.elu(_matmul(x, Wsk) + bsk)
    x1 = jax.nn.elu(_gat_layer(x, src, dst, W1, as1, ad1, b1, 4, 256))
    x2 = jax.nn.elu(_gat_layer(x1, src, dst, W2, as2, ad2, b2, 2, 256))
    combined = jnp.concatenate([identity, x2], axis=1)
    return _gat_layer(combined, src, dst, W3, as3, ad3, b3, 1, 128)
